# baseline (speedup 1.0000x reference)
import functools
import math

import jax
import jax.numpy as jnp
from jax import lax
from jax.experimental import pallas as pl
from jax.experimental.pallas import tpu as pltpu
from jax.experimental.pallas import tpu_sc as plsc

D = 128
B = 1024
L = 200
NTOK = B * L
NC, NS = 2, 16
NW = NC * NS
TPW = NTOK // NW
C = 40
NCHUNK = TPW // C
NG = D // 16

_SCALE = math.sqrt(float(D))
_EPS2 = 1e-5 / float(D)
_INVD = 1.0 / float(D)

_mesh = plsc.VectorSubcoreMesh(core_axis_name="c", subcore_axis_name="s")


def _allsum(v):
    for sh in (8, 4, 2, 1):
        idx = lax.iota(jnp.int32, 16) ^ sh
        v = v + v.at[idx].get(mode="promise_in_bounds")
    return v


@functools.partial(
    pl.kernel,
    out_type=jax.ShapeDtypeStruct((NTOK, D), jnp.float32),
    mesh=_mesh,
    scratch_types=[
        [[pltpu.VMEM((2 * C,), jnp.int32) for _ in range(6)]
         for _ in range(2)],
        [[pltpu.VMEM((C, D), jnp.float32) for _ in range(6)]
         for _ in range(2)],
        [pltpu.VMEM((C, D), jnp.float32) for _ in range(2)],
        pltpu.VMEM((L, D), jnp.float32),
        pltpu.VMEM((2, D), jnp.float32),
        [pltpu.SemaphoreType.DMA for _ in range(2)],
        [pltpu.SemaphoreType.DMA for _ in range(2)],
        [pltpu.SemaphoreType.DMA for _ in range(2)],
        pltpu.SemaphoreType.DMA,
    ],
)
def _harmony_embed(xt, c_t, d_t, s_t, a_t, t_t, b_t, pe_s, gb, out,
                   idxp, bufs2, acc2, pe_v, gb_v, gsems, wsems, isems, psem):
    tables = (c_t, d_t, s_t, a_t, t_t, b_t)
    wid = lax.axis_index("s") * NC + lax.axis_index("c")
    wbase = wid * TPW
    NP = NCHUNK // 2

    pltpu.async_copy(pe_s, pe_v, psem)
    pltpu.async_copy(gb, gb_v, psem)

    def idx_start(p, par):
        for t in range(6):
            pltpu.async_copy(
                xt.at[pl.ds(t * NTOK + wbase + p * 2 * C, 2 * C)],
                idxp[par][t], isems[par])

    def idx_wait(p, par):
        for t in range(6):
            pltpu.make_async_copy(
                xt.at[pl.ds(t * NTOK + wbase + p * 2 * C, 2 * C)],
                idxp[par][t], isems[par]).wait()

    def fire(b, ci, par, half):
        for t in range(6):
            pltpu.async_copy(
                tables[t].at[idxp[par][t].at[pl.ds(half * C, C)]],
                bufs2[b][t], gsems[b])

    def drain(b, par, half):
        for t in range(6):
            pltpu.make_async_copy(
                tables[t].at[idxp[par][t].at[pl.ds(half * C, C)]],
                bufs2[b][t], gsems[b]).wait()

    def compute(b, ci, gvs, bvs):
        l0 = lax.rem(ci * C, L)

        def row_body(r, carry):
            gvs, bvs = carry
            lr = l0 + r
            ys = []
            for c in range(NG):
                sl = pl.ds(c * 16, 16)
                a0 = bufs2[b][0][r, sl] + bufs2[b][1][r, sl]
                a1 = bufs2[b][2][r, sl] + bufs2[b][3][r, sl]
                a2 = bufs2[b][4][r, sl] + bufs2[b][5][r, sl]
                ys.append((a0 + a1) + a2 + pe_v[lr, sl])
            ts = ((ys[0] + ys[1]) + (ys[2] + ys[3])) + \
                 ((ys[4] + ys[5]) + (ys[6] + ys[7]))
            qs = [y * y for y in ys]
            tq = ((qs[0] + qs[1]) + (qs[2] + qs[3])) + \
                 ((qs[4] + qs[5]) + (qs[6] + qs[7]))
            sv = _allsum(ts)
            qv = _allsum(tq)
            mv = sv * _INVD
            wv = qv * _INVD - mv * mv + _EPS2
            rv = 2.2 / jnp.maximum(wv, 2.0)
            for _ in range(8):
                rv = rv * (1.5 - 0.5 * wv * rv * rv)
            for c in range(NG):
                sl = pl.ds(c * 16, 16)
                acc2[b][r, sl] = (ys[c] - mv) * rv * gvs[c] + bvs[c]
            return gvs, bvs

        lax.fori_loop(0, C, row_body, (gvs, bvs), unroll=2)

    def wb_wait(b, ci):
        pltpu.make_async_copy(
            acc2[b], out.at[pl.ds(wbase + ci * C, C), :], wsems[b]).wait()

    def wb_start(b, ci):
        pltpu.async_copy(
            acc2[b], out.at[pl.ds(wbase + ci * C, C), :], wsems[b])

    idx_start(0, 0)
    idx_wait(0, 0)
    fire(0, 0, 0, 0)
    pltpu.make_async_copy(pe_s, pe_v, psem).wait()
    pltpu.make_async_copy(gb, gb_v, psem).wait()
    gvs0 = tuple(gb_v[0, pl.ds(c * 16, 16)] for c in range(NG))
    bvs0 = tuple(gb_v[1, pl.ds(c * 16, 16)] for c in range(NG))

    def pair_fn(p, par, gvs, bvs):
        c0 = p * 2
        fire(1, c0 + 1, par, 1)

        @pl.when(p + 1 < NP)
        def _():
            idx_start(p + 1, par ^ 1)

        drain(0, par, 0)

        @pl.when(c0 >= 2)
        def _():
            wb_wait(0, c0 - 2)

        compute(0, c0, gvs, bvs)
        wb_start(0, c0)

        @pl.when(p + 1 < NP)
        def _():
            idx_wait(p + 1, par ^ 1)
            fire(0, c0 + 2, par ^ 1, 0)

        drain(1, par, 1)

        @pl.when(c0 + 1 >= 2)
        def _():
            wb_wait(1, c0 - 1)

        compute(1, c0 + 1, gvs, bvs)
        wb_start(1, c0 + 1)

    def super_body(qq, carry):
        gvs, bvs = carry
        pair_fn(qq * 2, 0, gvs, bvs)
        pair_fn(qq * 2 + 1, 1, gvs, bvs)
        return carry

    lax.fori_loop(0, NP // 2, super_body, (gvs0, bvs0))
    wb_wait(0, NCHUNK - 2)
    wb_wait(1, NCHUNK - 1)


def kernel(x, chord_table, dur_table, s_table, a_table, t_table, b_table,
           gamma, beta, pe):
    xt = jnp.transpose(x.reshape(NTOK, 6)).astype(jnp.int32).reshape(-1)
    pe_s = pe[:L] * jnp.float32(1.0 / _SCALE)
    gb = jnp.stack([gamma, beta]).astype(jnp.float32)
    out = _harmony_embed(xt, chord_table, dur_table, s_table, a_table,
                         t_table, b_table, pe_s, gb)
    return out.reshape(B, L, D)


# speedup vs baseline: 1.6304x; 1.6304x over previous
import functools
import math

import jax
import jax.numpy as jnp
from jax import lax
from jax.experimental import pallas as pl
from jax.experimental.pallas import tpu as pltpu
from jax.experimental.pallas import tpu_sc as plsc

D = 128
B = 1024
L = 200
NTOK = B * L
NC, NS = 2, 16
NW = NC * NS
TPW = NTOK // NW
C = 64
NCHUNK = TPW // C

_mesh = plsc.VectorSubcoreMesh(core_axis_name="c", subcore_axis_name="s")


@functools.partial(
    pl.kernel,
    out_type=jax.ShapeDtypeStruct((NTOK, D), jnp.float32),
    mesh=_mesh,
    scratch_types=[
        [pltpu.VMEM((6, 2 * C), jnp.int32) for _ in range(2)],
        [[pltpu.VMEM((C, D), jnp.float32) for _ in range(6)]
         for _ in range(2)],
        [pltpu.VMEM((C, D), jnp.float32) for _ in range(2)],
        [pltpu.SemaphoreType.DMA for _ in range(2)],
        [pltpu.SemaphoreType.DMA for _ in range(2)],
        [pltpu.SemaphoreType.DMA for _ in range(2)],
    ],
)
def _gather_sum(xt, c_t, d_t, s_t, a_t, t_t, b_t, out,
                idxp, bufs2, acc2, gsems, wsems, isems):
    tables = (c_t, d_t, s_t, a_t, t_t, b_t)
    wid = lax.axis_index("s") * NC + lax.axis_index("c")
    wbase = wid * TPW
    NP = NCHUNK // 2

    def idx_start(p, par):
        pltpu.async_copy(
            xt.at[:, pl.ds(wbase + p * 2 * C, 2 * C)], idxp[par], isems[par])

    def idx_wait(p, par):
        pltpu.make_async_copy(
            xt.at[:, pl.ds(wbase + p * 2 * C, 2 * C)], idxp[par],
            isems[par]).wait()

    def fire(b, ci, par, half):
        ib = idxp[par]
        for t in range(6):
            pltpu.async_copy(
                tables[t].at[ib.at[t, pl.ds(half * C, C)]],
                bufs2[b][t], gsems[b])

    def drain(b, par, half):
        ib = idxp[par]
        for t in range(6):
            pltpu.make_async_copy(
                tables[t].at[ib.at[t, pl.ds(half * C, C)]],
                bufs2[b][t], gsems[b]).wait()

    def compute(b):
        def row_body(r, rc):
            for c in range(D // 16):
                sl = pl.ds(c * 16, 16)
                a0 = bufs2[b][0][r, sl] + bufs2[b][1][r, sl]
                a1 = bufs2[b][2][r, sl] + bufs2[b][3][r, sl]
                a2 = bufs2[b][4][r, sl] + bufs2[b][5][r, sl]
                acc2[b][r, sl] = (a0 + a1) + a2
            return rc

        lax.fori_loop(0, C, row_body, 0, unroll=4)

    def wb_wait(b, ci):
        pltpu.make_async_copy(
            acc2[b], out.at[pl.ds(wbase + ci * C, C), :], wsems[b]).wait()

    def wb_start(b, ci):
        pltpu.async_copy(
            acc2[b], out.at[pl.ds(wbase + ci * C, C), :], wsems[b])

    idx_start(0, 0)
    idx_wait(0, 0)
    fire(0, 0, 0, 0)

    def pair_fn(p, par):
        c0 = p * 2
        fire(1, c0 + 1, par, 1)

        @pl.when(p + 1 < NP)
        def _():
            idx_start(p + 1, par ^ 1)

        drain(0, par, 0)

        @pl.when(c0 >= 2)
        def _():
            wb_wait(0, c0 - 2)

        compute(0)
        wb_start(0, c0)

        @pl.when(p + 1 < NP)
        def _():
            idx_wait(p + 1, par ^ 1)
            fire(0, c0 + 2, par ^ 1, 0)

        drain(1, par, 1)

        @pl.when(c0 + 1 >= 2)
        def _():
            wb_wait(1, c0 - 1)

        compute(1)
        wb_start(1, c0 + 1)

    def super_body(qq, carry):
        pair_fn(qq * 2, 0)
        pair_fn(qq * 2 + 1, 1)
        return carry

    lax.fori_loop(0, NP // 2, super_body, 0)
    wb_wait(0, NCHUNK - 2)
    wb_wait(1, NCHUNK - 1)


SEQ_BLK = 32
_SCALE = math.sqrt(float(D))


def _ln_body(c_ref, pe_ref, g_ref, b_ref, o_ref):
    y = c_ref[...] * _SCALE + pe_ref[...][None]
    mean = jnp.mean(y, axis=-1, keepdims=True)
    var = jnp.mean(jnp.square(y - mean), axis=-1, keepdims=True)
    o_ref[...] = (y - mean) * lax.rsqrt(var + 1e-5) * g_ref[...] + b_ref[...]


def kernel(x, chord_table, dur_table, s_table, a_table, t_table, b_table,
           gamma, beta, pe):
    xt = jnp.transpose(x.reshape(NTOK, 6)).astype(jnp.int32)
    combined = _gather_sum(xt, chord_table, dur_table, s_table, a_table,
                           t_table, b_table)
    comb3 = combined.reshape(B, L, D)
    pe200 = pe[:L]
    g2 = gamma.reshape(1, D)
    b2 = beta.reshape(1, D)
    out = pl.pallas_call(
        _ln_body,
        grid=(B // SEQ_BLK,),
        in_specs=[
            pl.BlockSpec((SEQ_BLK, L, D), lambda i: (i, 0, 0)),
            pl.BlockSpec((L, D), lambda i: (0, 0)),
            pl.BlockSpec((1, D), lambda i: (0, 0)),
            pl.BlockSpec((1, D), lambda i: (0, 0)),
        ],
        out_specs=pl.BlockSpec((SEQ_BLK, L, D), lambda i: (i, 0, 0)),
        out_shape=jax.ShapeDtypeStruct((B, L, D), jnp.float32),
    )(comb3, pe200, g2, b2)
    return out


# speedup vs baseline: 1.6631x; 1.0201x over previous
import functools
import math

import jax
import jax.numpy as jnp
from jax import lax
from jax.experimental import pallas as pl
from jax.experimental.pallas import tpu as pltpu
from jax.experimental.pallas import tpu_sc as plsc

D = 128
B = 1024
L = 200
NTOK = B * L
NC, NS = 2, 16
NW = NC * NS
TPW = NTOK // NW
C = 64
NCHUNK = TPW // C

_mesh = plsc.VectorSubcoreMesh(core_axis_name="c", subcore_axis_name="s")


@functools.partial(
    pl.kernel,
    out_type=jax.ShapeDtypeStruct((NTOK, D), jnp.float32),
    mesh=_mesh,
    scratch_types=[
        [pltpu.VMEM((6, 2 * C), jnp.int32) for _ in range(2)],
        [[pltpu.VMEM((C, D), jnp.float32) for _ in range(6)]
         for _ in range(2)],
        [pltpu.VMEM((C, D), jnp.float32) for _ in range(2)],
        [pltpu.SemaphoreType.DMA for _ in range(2)],
        [pltpu.SemaphoreType.DMA for _ in range(2)],
        [pltpu.SemaphoreType.DMA for _ in range(2)],
    ],
)
def _gather_sum(xt, c_t, d_t, s_t, a_t, t_t, b_t, out,
                idxp, bufs2, acc2, gsems, wsems, isems):
    tables = (c_t, d_t, s_t, a_t, t_t, b_t)
    wid = lax.axis_index("s") * NC + lax.axis_index("c")
    wbase = wid * TPW
    NP = NCHUNK // 2

    def idx_start(p, par):
        pltpu.async_copy(
            xt.at[:, pl.ds(wbase + p * 2 * C, 2 * C)], idxp[par], isems[par])

    def idx_wait(p, par):
        pltpu.make_async_copy(
            xt.at[:, pl.ds(wbase + p * 2 * C, 2 * C)], idxp[par],
            isems[par]).wait()

    def fire(b, ci, par, half):
        ib = idxp[par]
        for t in range(6):
            pltpu.async_copy(
                tables[t].at[ib.at[t, pl.ds(half * C, C)]],
                bufs2[b][t], gsems[b])

    def drain(b, par, half):
        ib = idxp[par]
        for t in range(6):
            pltpu.make_async_copy(
                tables[t].at[ib.at[t, pl.ds(half * C, C)]],
                bufs2[b][t], gsems[b]).wait()

    def compute(b):
        def row_body(r, rc):
            for c in range(D // 16):
                sl = pl.ds(c * 16, 16)
                a0 = bufs2[b][0][r, sl] + bufs2[b][1][r, sl]
                a1 = bufs2[b][2][r, sl] + bufs2[b][3][r, sl]
                a2 = bufs2[b][4][r, sl] + bufs2[b][5][r, sl]
                acc2[b][r, sl] = (a0 + a1) + a2
            return rc

        lax.fori_loop(0, C, row_body, 0, unroll=4)

    def wb_wait(b, ci):
        pltpu.make_async_copy(
            acc2[b], out.at[pl.ds(wbase + ci * C, C), :], wsems[b]).wait()

    def wb_start(b, ci):
        pltpu.async_copy(
            acc2[b], out.at[pl.ds(wbase + ci * C, C), :], wsems[b])

    idx_start(0, 0)
    idx_wait(0, 0)
    fire(0, 0, 0, 0)

    def pair_fn(p, par):
        c0 = p * 2
        fire(1, c0 + 1, par, 1)

        @pl.when(p + 1 < NP)
        def _():
            idx_start(p + 1, par ^ 1)

        drain(0, par, 0)

        @pl.when(c0 >= 2)
        def _():
            wb_wait(0, c0 - 2)

        compute(0)
        wb_start(0, c0)

        @pl.when(p + 1 < NP)
        def _():
            idx_wait(p + 1, par ^ 1)
            fire(0, c0 + 2, par ^ 1, 0)

        drain(1, par, 1)

        @pl.when(c0 + 1 >= 2)
        def _():
            wb_wait(1, c0 - 1)

        compute(1)
        wb_start(1, c0 + 1)

    def super_body(qq, carry):
        pair_fn(qq * 2, 0)
        pair_fn(qq * 2 + 1, 1)
        return carry

    lax.fori_loop(0, NP // 2, super_body, 0)
    wb_wait(0, NCHUNK - 2)
    wb_wait(1, NCHUNK - 1)


SEQ_BLK = 64
_SCALE = math.sqrt(float(D))


def _ln_body(c_ref, pe_ref, g_ref, b_ref, o_ref):
    y = c_ref[...] * _SCALE + pe_ref[...][None]
    mean = jnp.mean(y, axis=-1, keepdims=True)
    var = jnp.mean(jnp.square(y - mean), axis=-1, keepdims=True)
    o_ref[...] = (y - mean) * lax.rsqrt(var + 1e-5) * g_ref[...] + b_ref[...]


def kernel(x, chord_table, dur_table, s_table, a_table, t_table, b_table,
           gamma, beta, pe):
    xt = jnp.transpose(x.reshape(NTOK, 6)).astype(jnp.int32)
    combined = _gather_sum(xt, chord_table, dur_table, s_table, a_table,
                           t_table, b_table)
    comb3 = combined.reshape(B, L, D)
    pe200 = pe[:L]
    g2 = gamma.reshape(1, D)
    b2 = beta.reshape(1, D)
    out = pl.pallas_call(
        _ln_body,
        grid=(B // SEQ_BLK,),
        in_specs=[
            pl.BlockSpec((SEQ_BLK, L, D), lambda i: (i, 0, 0)),
            pl.BlockSpec((L, D), lambda i: (0, 0)),
            pl.BlockSpec((1, D), lambda i: (0, 0)),
            pl.BlockSpec((1, D), lambda i: (0, 0)),
        ],
        out_specs=pl.BlockSpec((SEQ_BLK, L, D), lambda i: (i, 0, 0)),
        out_shape=jax.ShapeDtypeStruct((B, L, D), jnp.float32),
    )(comb3, pe200, g2, b2)
    return out
